# Initial kernel scaffold; baseline (speedup 1.0000x reference)
import jax
import jax.numpy as jnp
from jax.experimental import pallas as pl


def kernel(x, edge_index, W_self1, W_neigh1, b1, W_self2, W_neigh2, b2):
    raise NotImplementedError("write your pallas kernel here")


# speedup vs baseline: 7.0466x; 7.0466x over previous
import functools

import jax
import jax.numpy as jnp
from jax import lax
from jax.experimental import pallas as pl
from jax.experimental.pallas import tpu as pltpu
from jax.experimental.pallas import tpu_sc as plsc

N = 50000
E = 800000

NC = 2
NS = 16

CB = 128
NROWS = 6256
ROWS_PAD = 6272

HI_CH, LO_CH = 392, 384
N_HI = 14
SCR = 8

RPS = 3136
NP = RPS * NS

_mesh = plsc.VectorSubcoreMesh(core_axis_name="c", subcore_axis_name="s")
_sc_params = pltpu.CompilerParams(use_tc_tiling_on_sc=False)


def _make_sc_segsum(D, with_deg):
  out_type = [jax.ShapeDtypeStruct((NC, NP, D), jnp.float32)]
  scratch = [
      pltpu.VMEM((SCR, CB), jnp.int32),
      pltpu.VMEM((SCR, CB), jnp.int32),
      pltpu.VMEM((CB, D), jnp.float32),
      pltpu.VMEM_SHARED((NP, D), jnp.float32),
      pltpu.SemaphoreType.DMA,
  ]
  if with_deg:
    out_type.append(jax.ShapeDtypeStruct((NP, 8), jnp.float32))
    scratch.append(pltpu.VMEM((CB, 8), jnp.float32))
    scratch.append(pltpu.VMEM_SHARED((NP, 8), jnp.float32))

  def body(p_hbm, src_hbm, dst_hbm, zf_hbm, zd_hbm, one_hbm,
           *outs_and_scratch):
    if with_deg:
      (agg_out, deg_out, sidx, didx, rows, acc_sh, sem, ones_v,
       deg_sh) = outs_and_scratch
    else:
      agg_out, sidx, didx, rows, acc_sh, sem = outs_and_scratch
    cid = lax.axis_index("c")
    sid = lax.axis_index("s")
    base = pl.multiple_of(sid * RPS, 8)
    pltpu.sync_copy(zf_hbm, acc_sh.at[pl.ds(base, RPS)])
    if with_deg:
      @pl.when(cid == 0)
      def _():
        pltpu.sync_copy(zd_hbm, deg_sh.at[pl.ds(base, RPS)])
        pltpu.sync_copy(one_hbm, ones_v)

    start = pl.multiple_of(
        jnp.where(sid < N_HI, HI_CH * sid,
                  HI_CH * N_HI + LO_CH * (sid - N_HI)), 8)
    nblk = jnp.where(sid < N_HI, HI_CH // SCR, LO_CH // SCR)
    plsc.subcore_barrier()

    def run(half, do_deg):
      def blk(b, carry):
        roff = pl.multiple_of(start + b * SCR, 8)
        pltpu.sync_copy(src_hbm.at[pl.ds(roff, SCR)], sidx)
        pltpu.sync_copy(dst_hbm.at[pl.ds(roff, SCR)], didx)
        for j in range(SCR):
          pltpu.async_copy(p_hbm.at[half].at[sidx.at[j]], rows, sem).wait()
          pltpu.sync_copy(rows, acc_sh.at[didx.at[j]], add=True)
          if do_deg:
            pltpu.sync_copy(ones_v, deg_sh.at[didx.at[j]], add=True)
        return carry
      lax.fori_loop(0, nblk, blk, 0)

    pl.when(cid == 0)(lambda: run(0, with_deg))
    pl.when(cid == 1)(lambda: run(1, False))

    plsc.subcore_barrier()
    pltpu.sync_copy(acc_sh.at[pl.ds(base, RPS)],
                    agg_out.at[cid, pl.ds(base, RPS)])
    if with_deg:
      @pl.when(cid == 0)
      def _():
        pltpu.sync_copy(deg_sh.at[pl.ds(base, RPS)],
                        deg_out.at[pl.ds(base, RPS)])

  return functools.partial(
      pl.kernel, out_type=out_type, mesh=_mesh, scratch_types=scratch,
      compiler_params=_sc_params, name=f"sc_segsum_d{D}")(body)


_sc_layer1 = _make_sc_segsum(16, with_deg=True)
_sc_layer2 = _make_sc_segsum(8, with_deg=False)

_BT = 2000
_GRID = (N // _BT,)


def _tc1_body(x_ref, wn_ref, ws_ref, b_ref, p_ref, s_ref):
  xb = x_ref[...]
  pr = jnp.dot(xb, wn_ref[...], preferred_element_type=jnp.float32)
  p_ref[0] = pr[:, :16]
  p_ref[1] = pr[:, 16:]
  s_ref[...] = (jnp.dot(xb, ws_ref[...], preferred_element_type=jnp.float32)
                + b_ref[...][None, :])


def _tc2_body(pa_ref, dg_ref, s1_ref, wn_ref, ws_ref, b_ref, p_ref, s_ref):
  agg = jnp.concatenate([pa_ref[0], pa_ref[1]], axis=1)
  deg = jnp.maximum(dg_ref[:, 0:1], 1.0)
  h = s1_ref[...] + agg / deg
  pr = jnp.dot(h, wn_ref[...], preferred_element_type=jnp.float32)
  p_ref[0] = pr[:, :8]
  p_ref[1] = pr[:, 8:]
  s_ref[...] = (jnp.dot(h, ws_ref[...], preferred_element_type=jnp.float32)
                + b_ref[...][None, :])


def _tc3_body(pa_ref, dg_ref, s2_ref, o_ref):
  agg = jnp.concatenate([pa_ref[0], pa_ref[1]], axis=1)
  deg = jnp.maximum(dg_ref[:, 0:1], 1.0)
  o_ref[...] = s2_ref[...] + agg / deg


def kernel(x, edge_index, W_self1, W_neigh1, b1, W_self2, W_neigh2, b2):
  src = edge_index[0].astype(jnp.int32)
  dst = edge_index[1].astype(jnp.int32)
  srcp = jnp.pad(src, (0, ROWS_PAD * CB - E)).reshape(ROWS_PAD, CB)
  dstp = jnp.pad(dst, (0, ROWS_PAD * CB - E),
                 constant_values=N).reshape(ROWS_PAD, CB)
  z16 = jnp.zeros((RPS, 16), jnp.float32)
  z8 = jnp.zeros((RPS, 8), jnp.float32)
  one8 = jnp.ones((CB, 8), jnp.float32)

  p1, s1 = pl.pallas_call(
      _tc1_body,
      grid=_GRID,
      in_specs=[
          pl.BlockSpec((_BT, 64), lambda i: (i, 0)),
          pl.BlockSpec((64, 32), lambda i: (0, 0)),
          pl.BlockSpec((64, 32), lambda i: (0, 0)),
          pl.BlockSpec((32,), lambda i: (0,)),
      ],
      out_specs=[pl.BlockSpec((2, _BT, 16), lambda i: (0, i, 0)),
                 pl.BlockSpec((_BT, 32), lambda i: (i, 0))],
      out_shape=[jax.ShapeDtypeStruct((2, N, 16), jnp.float32),
                 jax.ShapeDtypeStruct((N, 32), jnp.float32)],
  )(x, W_neigh1, W_self1, b1)

  agg1, deg = _sc_layer1(p1, srcp, dstp, z16, z8, one8)

  p2, s2 = pl.pallas_call(
      _tc2_body,
      grid=_GRID,
      in_specs=[
          pl.BlockSpec((2, _BT, 16), lambda i: (0, i, 0)),
          pl.BlockSpec((_BT, 8), lambda i: (i, 0)),
          pl.BlockSpec((_BT, 32), lambda i: (i, 0)),
          pl.BlockSpec((32, 16), lambda i: (0, 0)),
          pl.BlockSpec((32, 16), lambda i: (0, 0)),
          pl.BlockSpec((16,), lambda i: (0,)),
      ],
      out_specs=[pl.BlockSpec((2, _BT, 8), lambda i: (0, i, 0)),
                 pl.BlockSpec((_BT, 16), lambda i: (i, 0))],
      out_shape=[jax.ShapeDtypeStruct((2, N, 8), jnp.float32),
                 jax.ShapeDtypeStruct((N, 16), jnp.float32)],
  )(agg1, deg, s1, W_neigh2, W_self2, b2)

  (agg2,) = _sc_layer2(p2, srcp, dstp, z8, z8, one8)

  out = pl.pallas_call(
      _tc3_body,
      grid=_GRID,
      in_specs=[
          pl.BlockSpec((2, _BT, 8), lambda i: (0, i, 0)),
          pl.BlockSpec((_BT, 8), lambda i: (i, 0)),
          pl.BlockSpec((_BT, 16), lambda i: (i, 0)),
      ],
      out_specs=pl.BlockSpec((_BT, 16), lambda i: (i, 0)),
      out_shape=jax.ShapeDtypeStruct((N, 16), jnp.float32),
  )(agg2, deg, s2)

  return out


# speedup vs baseline: 12.4131x; 1.7616x over previous
import functools

import jax
import jax.numpy as jnp
from jax import lax
from jax.experimental import pallas as pl
from jax.experimental.pallas import tpu as pltpu
from jax.experimental.pallas import tpu_sc as plsc

N = 50000
E = 800000

NC = 2
NS = 16

Q = 512
NROWS = 1564
ROWS_PAD = 1568

HI_CH, LO_CH = 98, 96
N_HI = 14

RPS = 3128
NP = RPS * NS

_mesh = plsc.VectorSubcoreMesh(core_axis_name="c", subcore_axis_name="s")
_sc_params = pltpu.CompilerParams(use_tc_tiling_on_sc=False)


def _make_sc_segsum(D, with_deg):
  out_type = [jax.ShapeDtypeStruct((NC, NP, D), jnp.float32)]
  scratch = [
      pltpu.VMEM((Q,), jnp.int32),
      pltpu.VMEM((Q,), jnp.int32),
      pltpu.VMEM((Q,), jnp.int32),
      pltpu.VMEM((Q,), jnp.int32),
      pltpu.VMEM((Q, D), jnp.float32),
      pltpu.VMEM((Q, D), jnp.float32),
      pltpu.VMEM_SHARED((NP, D), jnp.float32),
      pltpu.SemaphoreType.DMA,
      pltpu.SemaphoreType.DMA,
  ]
  if with_deg:
    out_type.append(jax.ShapeDtypeStruct((NP, 8), jnp.float32))
    scratch.append(pltpu.VMEM((Q, 8), jnp.float32))
    scratch.append(pltpu.VMEM_SHARED((NP, 8), jnp.float32))

  def body(p_hbm, src_hbm, dst_hbm, zf_hbm, zd_hbm, one_hbm,
           *outs_and_scratch):
    if with_deg:
      (agg_out, deg_out, sidx0, didx0, sidx1, didx1, rows0, rows1,
       acc_sh, sem0, sem1, ones_v, deg_sh) = outs_and_scratch
    else:
      (agg_out, sidx0, didx0, sidx1, didx1, rows0, rows1,
       acc_sh, sem0, sem1) = outs_and_scratch
    cid = lax.axis_index("c")
    sid = lax.axis_index("s")
    base = pl.multiple_of(sid * RPS, 8)
    pltpu.sync_copy(zf_hbm, acc_sh.at[pl.ds(base, RPS)])
    if with_deg:
      @pl.when(cid == 0)
      def _():
        pltpu.sync_copy(zd_hbm, deg_sh.at[pl.ds(base, RPS)])
        pltpu.sync_copy(one_hbm, ones_v)

    start = jnp.where(sid < N_HI, HI_CH * sid,
                      HI_CH * N_HI + LO_CH * (sid - N_HI))
    count = jnp.where(sid < N_HI, HI_CH, LO_CH)
    last = start + count - 1
    plsc.subcore_barrier()

    def run(half, do_deg):
      p_half = p_hbm.at[half]

      def fire(sx, rw, sm, r):
        pltpu.sync_copy(src_hbm.at[r], sx)
        pltpu.async_copy(p_half.at[sx], rw, sm)

      def drain(sx, rw, sm, dx, r):
        pltpu.sync_copy(dst_hbm.at[r], dx)
        pltpu.make_async_copy(p_half.at[sx], rw, sm).wait()
        pltpu.sync_copy(rw, acc_sh.at[dx], add=True)
        if do_deg:
          pltpu.sync_copy(ones_v, deg_sh.at[dx], add=True)

      fire(sidx0, rows0, sem0, start)

      def pair(t, carry):
        a = start + 2 * t
        fire(sidx1, rows1, sem1, a + 1)
        drain(sidx0, rows0, sem0, didx0, a)
        fire(sidx0, rows0, sem0, jnp.minimum(a + 2, last))
        drain(sidx1, rows1, sem1, didx1, a + 1)
        return carry

      lax.fori_loop(0, count // 2, pair, 0)
      pltpu.make_async_copy(p_half.at[sidx0], rows0, sem0).wait()

    pl.when(cid == 0)(lambda: run(0, with_deg))
    pl.when(cid == 1)(lambda: run(1, False))

    plsc.subcore_barrier()
    pltpu.sync_copy(acc_sh.at[pl.ds(base, RPS)],
                    agg_out.at[cid, pl.ds(base, RPS)])
    if with_deg:
      @pl.when(cid == 0)
      def _():
        pltpu.sync_copy(deg_sh.at[pl.ds(base, RPS)],
                        deg_out.at[pl.ds(base, RPS)])

  return functools.partial(
      pl.kernel, out_type=out_type, mesh=_mesh, scratch_types=scratch,
      compiler_params=_sc_params, name=f"sc_segsum_d{D}")(body)


_sc_layer1 = _make_sc_segsum(16, with_deg=True)
_sc_layer2 = _make_sc_segsum(8, with_deg=False)

_BT = 2000
_GRID = (N // _BT,)


def _tc1_body(x_ref, wn_ref, ws_ref, b_ref, p_ref, s_ref):
  xb = x_ref[...]
  pr = jnp.dot(xb, wn_ref[...], preferred_element_type=jnp.float32)
  p_ref[0] = pr[:, :16]
  p_ref[1] = pr[:, 16:]
  s_ref[...] = (jnp.dot(xb, ws_ref[...], preferred_element_type=jnp.float32)
                + b_ref[...][None, :])


def _tc2_body(pa_ref, dg_ref, s1_ref, wn_ref, ws_ref, b_ref, p_ref, s_ref):
  agg = jnp.concatenate([pa_ref[0], pa_ref[1]], axis=1)
  deg = jnp.maximum(dg_ref[:, 0:1], 1.0)
  h = s1_ref[...] + agg / deg
  pr = jnp.dot(h, wn_ref[...], preferred_element_type=jnp.float32)
  p_ref[0] = pr[:, :8]
  p_ref[1] = pr[:, 8:]
  s_ref[...] = (jnp.dot(h, ws_ref[...], preferred_element_type=jnp.float32)
                + b_ref[...][None, :])


def _tc3_body(pa_ref, dg_ref, s2_ref, o_ref):
  agg = jnp.concatenate([pa_ref[0], pa_ref[1]], axis=1)
  deg = jnp.maximum(dg_ref[:, 0:1], 1.0)
  o_ref[...] = s2_ref[...] + agg / deg


def kernel(x, edge_index, W_self1, W_neigh1, b1, W_self2, W_neigh2, b2):
  src = edge_index[0].astype(jnp.int32)
  dst = edge_index[1].astype(jnp.int32)
  srcp = jnp.pad(src, (0, ROWS_PAD * Q - E)).reshape(ROWS_PAD, Q)
  dstp = jnp.pad(dst, (0, ROWS_PAD * Q - E),
                 constant_values=N).reshape(ROWS_PAD, Q)
  z16 = jnp.zeros((RPS, 16), jnp.float32)
  z8 = jnp.zeros((RPS, 8), jnp.float32)
  one8 = jnp.ones((Q, 8), jnp.float32)

  p1, s1 = pl.pallas_call(
      _tc1_body,
      grid=_GRID,
      in_specs=[
          pl.BlockSpec((_BT, 64), lambda i: (i, 0)),
          pl.BlockSpec((64, 32), lambda i: (0, 0)),
          pl.BlockSpec((64, 32), lambda i: (0, 0)),
          pl.BlockSpec((32,), lambda i: (0,)),
      ],
      out_specs=[pl.BlockSpec((2, _BT, 16), lambda i: (0, i, 0)),
                 pl.BlockSpec((_BT, 32), lambda i: (i, 0))],
      out_shape=[jax.ShapeDtypeStruct((2, N, 16), jnp.float32),
                 jax.ShapeDtypeStruct((N, 32), jnp.float32)],
  )(x, W_neigh1, W_self1, b1)

  agg1, deg = _sc_layer1(p1, srcp, dstp, z16, z8, one8)

  p2, s2 = pl.pallas_call(
      _tc2_body,
      grid=_GRID,
      in_specs=[
          pl.BlockSpec((2, _BT, 16), lambda i: (0, i, 0)),
          pl.BlockSpec((_BT, 8), lambda i: (i, 0)),
          pl.BlockSpec((_BT, 32), lambda i: (i, 0)),
          pl.BlockSpec((32, 16), lambda i: (0, 0)),
          pl.BlockSpec((32, 16), lambda i: (0, 0)),
          pl.BlockSpec((16,), lambda i: (0,)),
      ],
      out_specs=[pl.BlockSpec((2, _BT, 8), lambda i: (0, i, 0)),
                 pl.BlockSpec((_BT, 16), lambda i: (i, 0))],
      out_shape=[jax.ShapeDtypeStruct((2, N, 8), jnp.float32),
                 jax.ShapeDtypeStruct((N, 16), jnp.float32)],
  )(agg1, deg, s1, W_neigh2, W_self2, b2)

  (agg2,) = _sc_layer2(p2, srcp, dstp, z8, z8, one8)

  out = pl.pallas_call(
      _tc3_body,
      grid=_GRID,
      in_specs=[
          pl.BlockSpec((2, _BT, 8), lambda i: (0, i, 0)),
          pl.BlockSpec((_BT, 8), lambda i: (i, 0)),
          pl.BlockSpec((_BT, 16), lambda i: (i, 0)),
      ],
      out_specs=pl.BlockSpec((_BT, 16), lambda i: (i, 0)),
      out_shape=jax.ShapeDtypeStruct((N, 16), jnp.float32),
  )(agg2, deg, s2)

  return out


# speedup vs baseline: 13.6996x; 1.1036x over previous
import functools

import jax
import jax.numpy as jnp
from jax import lax
from jax.experimental import pallas as pl
from jax.experimental.pallas import tpu as pltpu
from jax.experimental.pallas import tpu_sc as plsc

N = 50000
E = 800000

NC = 2
NS = 16

Q = 512
QQ = 2048
NQUAD = 390
E_MAIN = NQUAD * QQ

HI_CH, LO_CH = 25, 24
N_HI = 6

RPS = 3128
NP = RPS * NS

_mesh = plsc.VectorSubcoreMesh(core_axis_name="c", subcore_axis_name="s")
_sc_params = pltpu.CompilerParams(use_tc_tiling_on_sc=False)


def _make_sc_segsum(D, with_deg):
  out_type = [jax.ShapeDtypeStruct((NC, NP, D), jnp.float32)]
  scratch = [
      pltpu.VMEM((QQ,), jnp.int32),
      pltpu.VMEM((QQ,), jnp.int32),
      pltpu.VMEM((Q, D), jnp.float32),
      pltpu.VMEM((Q, D), jnp.float32),
      pltpu.VMEM_SHARED((NP, D), jnp.float32),
      pltpu.SemaphoreType.DMA,
      pltpu.SemaphoreType.DMA,
  ]
  if with_deg:
    out_type.append(jax.ShapeDtypeStruct((NP, 8), jnp.float32))
    scratch.append(pltpu.VMEM((Q, 8), jnp.float32))
    scratch.append(pltpu.VMEM_SHARED((NP, 8), jnp.float32))

  def body(p_hbm, ei_hbm, tail_hbm, zf_hbm, zd_hbm, one_hbm,
           *outs_and_scratch):
    if with_deg:
      (agg_out, deg_out, sidx, didx, rows0, rows1,
       acc_sh, sem0, sem1, ones_v, deg_sh) = outs_and_scratch
    else:
      (agg_out, sidx, didx, rows0, rows1,
       acc_sh, sem0, sem1) = outs_and_scratch
    cid = lax.axis_index("c")
    sid = lax.axis_index("s")
    base = pl.multiple_of(sid * RPS, 8)
    pltpu.sync_copy(zf_hbm, acc_sh.at[pl.ds(base, RPS)])
    if with_deg:
      @pl.when(cid == 0)
      def _():
        pltpu.sync_copy(zd_hbm, deg_sh.at[pl.ds(base, RPS)])
        pltpu.sync_copy(one_hbm, ones_v)

    qstart = jnp.where(sid < N_HI, HI_CH * sid,
                       HI_CH * N_HI + LO_CH * (sid - N_HI))
    qcount = jnp.where(sid < N_HI, HI_CH, LO_CH)
    plsc.subcore_barrier()

    def run(half, do_deg):
      p_half = p_hbm.at[half]
      sx = [sidx.at[pl.ds(k * Q, Q)] for k in range(4)]
      dx = [didx.at[pl.ds(k * Q, Q)] for k in range(4)]
      rw = [rows0, rows1]
      sm = [sem0, sem1]

      def quad_body():
        pltpu.async_copy(p_half.at[sx[0]], rows0, sem0)
        pltpu.async_copy(p_half.at[sx[1]], rows1, sem1)
        for k in range(4):
          pltpu.make_async_copy(p_half.at[sx[k]], rw[k % 2],
                                sm[k % 2]).wait()
          pltpu.sync_copy(rw[k % 2], acc_sh.at[dx[k]], add=True)
          if k + 2 < 4:
            pltpu.async_copy(p_half.at[sx[k + 2]], rw[k % 2], sm[k % 2])
          if do_deg:
            pltpu.sync_copy(ones_v, deg_sh.at[dx[k]], add=True)

      def quad(t, carry):
        off = pl.multiple_of((qstart + t) * QQ, 8)
        pltpu.sync_copy(ei_hbm.at[0, pl.ds(off, QQ)], sidx)
        pltpu.sync_copy(ei_hbm.at[1, pl.ds(off, QQ)], didx)
        quad_body()
        return carry

      lax.fori_loop(0, qcount, quad, 0)

      @pl.when(sid == NS - 1)
      def _():
        pltpu.sync_copy(tail_hbm.at[0], sidx)
        pltpu.sync_copy(tail_hbm.at[1], didx)
        quad_body()

    pl.when(cid == 0)(lambda: run(0, with_deg))
    pl.when(cid == 1)(lambda: run(1, False))

    plsc.subcore_barrier()
    pltpu.sync_copy(acc_sh.at[pl.ds(base, RPS)],
                    agg_out.at[cid, pl.ds(base, RPS)])
    if with_deg:
      @pl.when(cid == 0)
      def _():
        pltpu.sync_copy(deg_sh.at[pl.ds(base, RPS)],
                        deg_out.at[pl.ds(base, RPS)])

  return functools.partial(
      pl.kernel, out_type=out_type, mesh=_mesh, scratch_types=scratch,
      compiler_params=_sc_params, name=f"sc_segsum_d{D}")(body)


_sc_layer1 = _make_sc_segsum(16, with_deg=True)
_sc_layer2 = _make_sc_segsum(8, with_deg=False)

_BT = 2000
_GRID = (N // _BT,)


def _tc1_body(x_ref, wn_ref, ws_ref, b_ref, p_ref, s_ref):
  xb = x_ref[...]
  pr = jnp.dot(xb, wn_ref[...], preferred_element_type=jnp.float32)
  p_ref[0] = pr[:, :16]
  p_ref[1] = pr[:, 16:]
  s_ref[...] = (jnp.dot(xb, ws_ref[...], preferred_element_type=jnp.float32)
                + b_ref[...][None, :])


def _tc2_body(pa_ref, dg_ref, s1_ref, wn_ref, ws_ref, b_ref, p_ref, s_ref):
  agg = jnp.concatenate([pa_ref[0], pa_ref[1]], axis=1)
  deg = jnp.maximum(dg_ref[:, 0:1], 1.0)
  h = s1_ref[...] + agg / deg
  pr = jnp.dot(h, wn_ref[...], preferred_element_type=jnp.float32)
  p_ref[0] = pr[:, :8]
  p_ref[1] = pr[:, 8:]
  s_ref[...] = (jnp.dot(h, ws_ref[...], preferred_element_type=jnp.float32)
                + b_ref[...][None, :])


def _tc3_body(pa_ref, dg_ref, s2_ref, o_ref):
  agg = jnp.concatenate([pa_ref[0], pa_ref[1]], axis=1)
  deg = jnp.maximum(dg_ref[:, 0:1], 1.0)
  o_ref[...] = s2_ref[...] + agg / deg


def kernel(x, edge_index, W_self1, W_neigh1, b1, W_self2, W_neigh2, b2):
  ei = edge_index.astype(jnp.int32)
  pad = jnp.stack([jnp.zeros((QQ - (E - E_MAIN),), jnp.int32),
                   jnp.full((QQ - (E - E_MAIN),), N, jnp.int32)])
  tail = jnp.concatenate([ei[:, E_MAIN:], pad], axis=1)
  z16 = jnp.zeros((RPS, 16), jnp.float32)
  z8 = jnp.zeros((RPS, 8), jnp.float32)
  one8 = jnp.ones((Q, 8), jnp.float32)

  p1, s1 = pl.pallas_call(
      _tc1_body,
      grid=_GRID,
      in_specs=[
          pl.BlockSpec((_BT, 64), lambda i: (i, 0)),
          pl.BlockSpec((64, 32), lambda i: (0, 0)),
          pl.BlockSpec((64, 32), lambda i: (0, 0)),
          pl.BlockSpec((32,), lambda i: (0,)),
      ],
      out_specs=[pl.BlockSpec((2, _BT, 16), lambda i: (0, i, 0)),
                 pl.BlockSpec((_BT, 32), lambda i: (i, 0))],
      out_shape=[jax.ShapeDtypeStruct((2, N, 16), jnp.float32),
                 jax.ShapeDtypeStruct((N, 32), jnp.float32)],
  )(x, W_neigh1, W_self1, b1)

  agg1, deg = _sc_layer1(p1, ei, tail, z16, z8, one8)

  p2, s2 = pl.pallas_call(
      _tc2_body,
      grid=_GRID,
      in_specs=[
          pl.BlockSpec((2, _BT, 16), lambda i: (0, i, 0)),
          pl.BlockSpec((_BT, 8), lambda i: (i, 0)),
          pl.BlockSpec((_BT, 32), lambda i: (i, 0)),
          pl.BlockSpec((32, 16), lambda i: (0, 0)),
          pl.BlockSpec((32, 16), lambda i: (0, 0)),
          pl.BlockSpec((16,), lambda i: (0,)),
      ],
      out_specs=[pl.BlockSpec((2, _BT, 8), lambda i: (0, i, 0)),
                 pl.BlockSpec((_BT, 16), lambda i: (i, 0))],
      out_shape=[jax.ShapeDtypeStruct((2, N, 8), jnp.float32),
                 jax.ShapeDtypeStruct((N, 16), jnp.float32)],
  )(agg1, deg, s1, W_neigh2, W_self2, b2)

  (agg2,) = _sc_layer2(p2, ei, tail, z8, z8, one8)

  out = pl.pallas_call(
      _tc3_body,
      grid=_GRID,
      in_specs=[
          pl.BlockSpec((2, _BT, 8), lambda i: (0, i, 0)),
          pl.BlockSpec((_BT, 8), lambda i: (i, 0)),
          pl.BlockSpec((_BT, 16), lambda i: (i, 0)),
      ],
      out_specs=pl.BlockSpec((_BT, 16), lambda i: (i, 0)),
      out_shape=jax.ShapeDtypeStruct((N, 16), jnp.float32),
  )(agg2, deg, s2)

  return out


# speedup vs baseline: 13.9570x; 1.0188x over previous
import functools

import jax
import jax.numpy as jnp
from jax import lax
from jax.experimental import pallas as pl
from jax.experimental.pallas import tpu as pltpu
from jax.experimental.pallas import tpu_sc as plsc

N = 50000
E = 800000

NC = 2
NS = 16

Q = 512
QQ = 2048
NQUAD = 390
E_MAIN = NQUAD * QQ

HI_CH, LO_CH = 25, 24
N_HI = 6

RPS = 3128
NP = RPS * NS

_mesh = plsc.VectorSubcoreMesh(core_axis_name="c", subcore_axis_name="s")
_sc_params = pltpu.CompilerParams(use_tc_tiling_on_sc=False)


def _make_sc_segsum(D, with_deg):
  out_type = [jax.ShapeDtypeStruct((NC, NP, D), jnp.float32)]
  scratch = [
      pltpu.VMEM((QQ,), jnp.int32),
      pltpu.VMEM((QQ,), jnp.int32),
      pltpu.VMEM((Q, D), jnp.float32),
      pltpu.VMEM((Q, D), jnp.float32),
      pltpu.VMEM((Q, D), jnp.float32),
      pltpu.VMEM_SHARED((NP, D), jnp.float32),
      pltpu.SemaphoreType.DMA,
      pltpu.SemaphoreType.DMA,
      pltpu.SemaphoreType.DMA,
  ]
  if with_deg:
    out_type.append(jax.ShapeDtypeStruct((NP, 8), jnp.float32))
    scratch.append(pltpu.VMEM((Q, 8), jnp.float32))
    scratch.append(pltpu.VMEM_SHARED((NP, 8), jnp.float32))

  def body(p_hbm, ei_hbm, tail_hbm, zf_hbm, zd_hbm, one_hbm,
           *outs_and_scratch):
    if with_deg:
      (agg_out, deg_out, sidx, didx, rows0, rows1, rows2,
       acc_sh, sem0, sem1, sem2, ones_v, deg_sh) = outs_and_scratch
    else:
      (agg_out, sidx, didx, rows0, rows1, rows2,
       acc_sh, sem0, sem1, sem2) = outs_and_scratch
    cid = lax.axis_index("c")
    sid = lax.axis_index("s")
    base = pl.multiple_of(sid * RPS, 8)
    pltpu.sync_copy(zf_hbm, acc_sh.at[pl.ds(base, RPS)])
    if with_deg:
      @pl.when(cid == 0)
      def _():
        pltpu.sync_copy(zd_hbm, deg_sh.at[pl.ds(base, RPS)])
        pltpu.sync_copy(one_hbm, ones_v)

    qstart = jnp.where(sid < N_HI, HI_CH * sid,
                       HI_CH * N_HI + LO_CH * (sid - N_HI))
    qcount = jnp.where(sid < N_HI, HI_CH, LO_CH)
    plsc.subcore_barrier()

    def run(half, do_deg):
      p_half = p_hbm.at[half]
      sx = [sidx.at[pl.ds(k * Q, Q)] for k in range(4)]
      dx = [didx.at[pl.ds(k * Q, Q)] for k in range(4)]
      rw = [rows0, rows1, rows2]
      sm = [sem0, sem1, sem2]

      def quad_body():
        pltpu.async_copy(p_half.at[sx[0]], rows0, sem0)
        pltpu.async_copy(p_half.at[sx[1]], rows1, sem1)
        pltpu.async_copy(p_half.at[sx[2]], rows2, sem2)
        for k in range(4):
          pltpu.make_async_copy(p_half.at[sx[k]], rw[k % 3],
                                sm[k % 3]).wait()
          pltpu.sync_copy(rw[k % 3], acc_sh.at[dx[k]], add=True)
          if k + 3 < 4:
            pltpu.async_copy(p_half.at[sx[k + 3]], rw[k % 3], sm[k % 3])
          if do_deg:
            pltpu.sync_copy(ones_v, deg_sh.at[dx[k]], add=True)

      def quad(t, carry):
        off = pl.multiple_of((qstart + t) * QQ, 8)
        pltpu.sync_copy(ei_hbm.at[0, pl.ds(off, QQ)], sidx)
        pltpu.sync_copy(ei_hbm.at[1, pl.ds(off, QQ)], didx)
        quad_body()
        return carry

      lax.fori_loop(0, qcount, quad, 0)

      @pl.when(sid == NS - 1)
      def _():
        pltpu.sync_copy(tail_hbm.at[0], sidx)
        pltpu.sync_copy(tail_hbm.at[1], didx)
        quad_body()

    pl.when(cid == 0)(lambda: run(0, with_deg))
    pl.when(cid == 1)(lambda: run(1, False))

    plsc.subcore_barrier()
    pltpu.sync_copy(acc_sh.at[pl.ds(base, RPS)],
                    agg_out.at[cid, pl.ds(base, RPS)])
    if with_deg:
      @pl.when(cid == 0)
      def _():
        pltpu.sync_copy(deg_sh.at[pl.ds(base, RPS)],
                        deg_out.at[pl.ds(base, RPS)])

  return functools.partial(
      pl.kernel, out_type=out_type, mesh=_mesh, scratch_types=scratch,
      compiler_params=_sc_params, name=f"sc_segsum_d{D}")(body)


_sc_layer1 = _make_sc_segsum(16, with_deg=True)
_sc_layer2 = _make_sc_segsum(8, with_deg=False)

_BT = 5000
_GRID = (N // _BT,)


def _tc1_body(x_ref, wn_ref, ws_ref, b_ref, p_ref, s_ref):
  xb = x_ref[...]
  pr = jnp.dot(xb, wn_ref[...], preferred_element_type=jnp.float32)
  p_ref[0] = pr[:, :16]
  p_ref[1] = pr[:, 16:]
  s_ref[...] = (jnp.dot(xb, ws_ref[...], preferred_element_type=jnp.float32)
                + b_ref[...][None, :])


def _tc2_body(pa_ref, dg_ref, s1_ref, wn_ref, ws_ref, b_ref, p_ref, s_ref):
  agg = jnp.concatenate([pa_ref[0], pa_ref[1]], axis=1)
  deg = jnp.maximum(dg_ref[:, 0:1], 1.0)
  h = s1_ref[...] + agg / deg
  pr = jnp.dot(h, wn_ref[...], preferred_element_type=jnp.float32)
  p_ref[0] = pr[:, :8]
  p_ref[1] = pr[:, 8:]
  s_ref[...] = (jnp.dot(h, ws_ref[...], preferred_element_type=jnp.float32)
                + b_ref[...][None, :])


def _tc3_body(pa_ref, dg_ref, s2_ref, o_ref):
  agg = jnp.concatenate([pa_ref[0], pa_ref[1]], axis=1)
  deg = jnp.maximum(dg_ref[:, 0:1], 1.0)
  o_ref[...] = s2_ref[...] + agg / deg


def kernel(x, edge_index, W_self1, W_neigh1, b1, W_self2, W_neigh2, b2):
  ei = edge_index.astype(jnp.int32)
  pad = jnp.stack([jnp.zeros((QQ - (E - E_MAIN),), jnp.int32),
                   jnp.full((QQ - (E - E_MAIN),), N, jnp.int32)])
  tail = jnp.concatenate([ei[:, E_MAIN:], pad], axis=1)
  z16 = jnp.zeros((RPS, 16), jnp.float32)
  z8 = jnp.zeros((RPS, 8), jnp.float32)
  one8 = jnp.ones((Q, 8), jnp.float32)

  p1, s1 = pl.pallas_call(
      _tc1_body,
      grid=_GRID,
      in_specs=[
          pl.BlockSpec((_BT, 64), lambda i: (i, 0)),
          pl.BlockSpec((64, 32), lambda i: (0, 0)),
          pl.BlockSpec((64, 32), lambda i: (0, 0)),
          pl.BlockSpec((32,), lambda i: (0,)),
      ],
      out_specs=[pl.BlockSpec((2, _BT, 16), lambda i: (0, i, 0)),
                 pl.BlockSpec((_BT, 32), lambda i: (i, 0))],
      out_shape=[jax.ShapeDtypeStruct((2, N, 16), jnp.float32),
                 jax.ShapeDtypeStruct((N, 32), jnp.float32)],
  )(x, W_neigh1, W_self1, b1)

  agg1, deg = _sc_layer1(p1, ei, tail, z16, z8, one8)

  p2, s2 = pl.pallas_call(
      _tc2_body,
      grid=_GRID,
      in_specs=[
          pl.BlockSpec((2, _BT, 16), lambda i: (0, i, 0)),
          pl.BlockSpec((_BT, 8), lambda i: (i, 0)),
          pl.BlockSpec((_BT, 32), lambda i: (i, 0)),
          pl.BlockSpec((32, 16), lambda i: (0, 0)),
          pl.BlockSpec((32, 16), lambda i: (0, 0)),
          pl.BlockSpec((16,), lambda i: (0,)),
      ],
      out_specs=[pl.BlockSpec((2, _BT, 8), lambda i: (0, i, 0)),
                 pl.BlockSpec((_BT, 16), lambda i: (i, 0))],
      out_shape=[jax.ShapeDtypeStruct((2, N, 8), jnp.float32),
                 jax.ShapeDtypeStruct((N, 16), jnp.float32)],
  )(agg1, deg, s1, W_neigh2, W_self2, b2)

  (agg2,) = _sc_layer2(p2, ei, tail, z8, z8, one8)

  out = pl.pallas_call(
      _tc3_body,
      grid=_GRID,
      in_specs=[
          pl.BlockSpec((2, _BT, 8), lambda i: (0, i, 0)),
          pl.BlockSpec((_BT, 8), lambda i: (i, 0)),
          pl.BlockSpec((_BT, 16), lambda i: (i, 0)),
      ],
      out_specs=pl.BlockSpec((_BT, 16), lambda i: (i, 0)),
      out_shape=jax.ShapeDtypeStruct((N, 16), jnp.float32),
  )(agg2, deg, s2)

  return out


# speedup vs baseline: 14.3126x; 1.0255x over previous
import functools

import jax
import jax.numpy as jnp
from jax import lax
from jax.experimental import pallas as pl
from jax.experimental.pallas import tpu as pltpu
from jax.experimental.pallas import tpu_sc as plsc

N = 50000
E = 800000

NC = 2
NS = 16

Q = 512
QQ = 2048
NQUAD = 390
E_MAIN = NQUAD * QQ

HI_CH, LO_CH = 25, 24
N_HI = 6

RPS = 3128
NP = RPS * NS

_mesh = plsc.VectorSubcoreMesh(core_axis_name="c", subcore_axis_name="s")
_sc_params = pltpu.CompilerParams(use_tc_tiling_on_sc=False)


def _make_sc_segsum(D, with_deg):
  out_type = [jax.ShapeDtypeStruct((NC, NP, D), jnp.float32)]
  scratch = [
      pltpu.VMEM((QQ,), jnp.int32),
      pltpu.VMEM((QQ,), jnp.int32),
      pltpu.VMEM((Q, D), jnp.float32),
      pltpu.VMEM((Q, D), jnp.float32),
      pltpu.VMEM((Q, D), jnp.float32),
      pltpu.VMEM((Q, D), jnp.float32),
      pltpu.VMEM_SHARED((NP, D), jnp.float32),
      pltpu.SemaphoreType.DMA,
      pltpu.SemaphoreType.DMA,
      pltpu.SemaphoreType.DMA,
      pltpu.SemaphoreType.DMA,
      pltpu.SemaphoreType.DMA,
  ]
  if with_deg:
    out_type.append(jax.ShapeDtypeStruct((NP, 8), jnp.float32))
    scratch.append(pltpu.VMEM((Q, 8), jnp.float32))
    scratch.append(pltpu.VMEM_SHARED((NP, 8), jnp.float32))

  def body(p_hbm, ei_hbm, tail_hbm, zf_hbm, zd_hbm, one_hbm,
           *outs_and_scratch):
    if with_deg:
      (agg_out, deg_out, sidx, didx, r0, r1, r2, r3,
       acc_sh, g0, g1, g2, g3, sem_s, ones_v, deg_sh) = outs_and_scratch
    else:
      (agg_out, sidx, didx, r0, r1, r2, r3,
       acc_sh, g0, g1, g2, g3, sem_s) = outs_and_scratch
    cid = lax.axis_index("c")
    sid = lax.axis_index("s")
    base = pl.multiple_of(sid * RPS, 8)
    pltpu.sync_copy(zf_hbm, acc_sh.at[pl.ds(base, RPS)])
    if with_deg:
      @pl.when(cid == 0)
      def _():
        pltpu.sync_copy(zd_hbm, deg_sh.at[pl.ds(base, RPS)])
        pltpu.sync_copy(one_hbm, ones_v)

    qstart = jnp.where(sid < N_HI, HI_CH * sid,
                       HI_CH * N_HI + LO_CH * (sid - N_HI))
    qcount = jnp.where(sid < N_HI, HI_CH, LO_CH)
    plsc.subcore_barrier()

    def run(half, do_deg):
      p_half = p_hbm.at[half]
      sx = [sidx.at[pl.ds(k * Q, Q)] for k in range(4)]
      dx = [didx.at[pl.ds(k * Q, Q)] for k in range(4)]
      rw = [r0, r1, r2, r3]
      gs = [g0, g1, g2, g3]

      def drain_scatters():
        for k in range(4):
          pltpu.make_async_copy(rw[k], acc_sh.at[dx[k]], sem_s).wait()
          if do_deg:
            pltpu.make_async_copy(ones_v, deg_sh.at[dx[k]], sem_s).wait()

      def quad_body():
        for k in range(4):
          pltpu.async_copy(p_half.at[sx[k]], rw[k], gs[k])
        for k in range(4):
          pltpu.make_async_copy(p_half.at[sx[k]], rw[k], gs[k]).wait()
          pltpu.async_copy(rw[k], acc_sh.at[dx[k]], sem_s, add=True)
          if do_deg:
            pltpu.async_copy(ones_v, deg_sh.at[dx[k]], sem_s, add=True)

      def quad(t, carry):
        @pl.when(t > 0)
        def _():
          drain_scatters()
        off = pl.multiple_of((qstart + t) * QQ, 8)
        pltpu.sync_copy(ei_hbm.at[0, pl.ds(off, QQ)], sidx)
        pltpu.sync_copy(ei_hbm.at[1, pl.ds(off, QQ)], didx)
        quad_body()
        return carry

      lax.fori_loop(0, qcount, quad, 0)
      drain_scatters()

      @pl.when(sid == NS - 1)
      def _():
        pltpu.sync_copy(tail_hbm.at[0], sidx)
        pltpu.sync_copy(tail_hbm.at[1], didx)
        quad_body()
        drain_scatters()

    pl.when(cid == 0)(lambda: run(0, with_deg))
    pl.when(cid == 1)(lambda: run(1, False))

    plsc.subcore_barrier()
    pltpu.sync_copy(acc_sh.at[pl.ds(base, RPS)],
                    agg_out.at[cid, pl.ds(base, RPS)])
    if with_deg:
      @pl.when(cid == 0)
      def _():
        pltpu.sync_copy(deg_sh.at[pl.ds(base, RPS)],
                        deg_out.at[pl.ds(base, RPS)])

  return functools.partial(
      pl.kernel, out_type=out_type, mesh=_mesh, scratch_types=scratch,
      compiler_params=_sc_params, name=f"sc_segsum_d{D}")(body)


_sc_layer1 = _make_sc_segsum(16, with_deg=True)
_sc_layer2 = _make_sc_segsum(8, with_deg=False)

_BT = 5000
_GRID = (N // _BT,)


def _tc1_body(x_ref, wn_ref, ws_ref, b_ref, p_ref, s_ref):
  xb = x_ref[...]
  pr = jnp.dot(xb, wn_ref[...], preferred_element_type=jnp.float32)
  p_ref[0] = pr[:, :16]
  p_ref[1] = pr[:, 16:]
  s_ref[...] = (jnp.dot(xb, ws_ref[...], preferred_element_type=jnp.float32)
                + b_ref[...][None, :])


def _tc2_body(pa_ref, dg_ref, s1_ref, wn_ref, ws_ref, b_ref, p_ref, s_ref):
  agg = jnp.concatenate([pa_ref[0], pa_ref[1]], axis=1)
  deg = jnp.maximum(dg_ref[:, 0:1], 1.0)
  h = s1_ref[...] + agg / deg
  pr = jnp.dot(h, wn_ref[...], preferred_element_type=jnp.float32)
  p_ref[0] = pr[:, :8]
  p_ref[1] = pr[:, 8:]
  s_ref[...] = (jnp.dot(h, ws_ref[...], preferred_element_type=jnp.float32)
                + b_ref[...][None, :])


def _tc3_body(pa_ref, dg_ref, s2_ref, o_ref):
  agg = jnp.concatenate([pa_ref[0], pa_ref[1]], axis=1)
  deg = jnp.maximum(dg_ref[:, 0:1], 1.0)
  o_ref[...] = s2_ref[...] + agg / deg


def kernel(x, edge_index, W_self1, W_neigh1, b1, W_self2, W_neigh2, b2):
  ei = edge_index.astype(jnp.int32)
  pad = jnp.stack([jnp.zeros((QQ - (E - E_MAIN),), jnp.int32),
                   jnp.full((QQ - (E - E_MAIN),), N, jnp.int32)])
  tail = jnp.concatenate([ei[:, E_MAIN:], pad], axis=1)
  z16 = jnp.zeros((RPS, 16), jnp.float32)
  z8 = jnp.zeros((RPS, 8), jnp.float32)
  one8 = jnp.ones((Q, 8), jnp.float32)

  p1, s1 = pl.pallas_call(
      _tc1_body,
      grid=_GRID,
      in_specs=[
          pl.BlockSpec((_BT, 64), lambda i: (i, 0)),
          pl.BlockSpec((64, 32), lambda i: (0, 0)),
          pl.BlockSpec((64, 32), lambda i: (0, 0)),
          pl.BlockSpec((32,), lambda i: (0,)),
      ],
      out_specs=[pl.BlockSpec((2, _BT, 16), lambda i: (0, i, 0)),
                 pl.BlockSpec((_BT, 32), lambda i: (i, 0))],
      out_shape=[jax.ShapeDtypeStruct((2, N, 16), jnp.float32),
                 jax.ShapeDtypeStruct((N, 32), jnp.float32)],
  )(x, W_neigh1, W_self1, b1)

  agg1, deg = _sc_layer1(p1, ei, tail, z16, z8, one8)

  p2, s2 = pl.pallas_call(
      _tc2_body,
      grid=_GRID,
      in_specs=[
          pl.BlockSpec((2, _BT, 16), lambda i: (0, i, 0)),
          pl.BlockSpec((_BT, 8), lambda i: (i, 0)),
          pl.BlockSpec((_BT, 32), lambda i: (i, 0)),
          pl.BlockSpec((32, 16), lambda i: (0, 0)),
          pl.BlockSpec((32, 16), lambda i: (0, 0)),
          pl.BlockSpec((16,), lambda i: (0,)),
      ],
      out_specs=[pl.BlockSpec((2, _BT, 8), lambda i: (0, i, 0)),
                 pl.BlockSpec((_BT, 16), lambda i: (i, 0))],
      out_shape=[jax.ShapeDtypeStruct((2, N, 8), jnp.float32),
                 jax.ShapeDtypeStruct((N, 16), jnp.float32)],
  )(agg1, deg, s1, W_neigh2, W_self2, b2)

  (agg2,) = _sc_layer2(p2, ei, tail, z8, z8, one8)

  out = pl.pallas_call(
      _tc3_body,
      grid=_GRID,
      in_specs=[
          pl.BlockSpec((2, _BT, 8), lambda i: (0, i, 0)),
          pl.BlockSpec((_BT, 8), lambda i: (i, 0)),
          pl.BlockSpec((_BT, 16), lambda i: (i, 0)),
      ],
      out_specs=pl.BlockSpec((_BT, 16), lambda i: (i, 0)),
      out_shape=jax.ShapeDtypeStruct((N, 16), jnp.float32),
  )(agg2, deg, s2)

  return out
